# baseline (speedup 1.0000x reference)
import functools

import jax
import jax.numpy as jnp
from jax import lax
from jax.experimental import pallas as pl
from jax.experimental.pallas import tpu as pltpu
from jax.experimental.pallas import tpu_sc as plsc

NUM_CLASSES = 1000000
EMBED_DIM = 32
BATCH = 16384
FIELDS = 26

NC = 2
NS = 16
NW = NC * NS
PER_B = BATCH // NW
CHUNK_B = 8
N_CHUNKS = PER_B // CHUNK_B
NBUF = 2
N_MAIN = N_CHUNKS // NBUF - 1

assert PER_B * NW == BATCH
assert CHUNK_B * N_CHUNKS == PER_B
assert N_CHUNKS % NBUF == 0


def _make_gather():
    mesh = plsc.VectorSubcoreMesh(core_axis_name="c", subcore_axis_name="s")

    @functools.partial(
        pl.kernel,
        mesh=mesh,
        out_type=jax.ShapeDtypeStruct((BATCH, FIELDS, EMBED_DIM), jnp.float32),
        scratch_types=[
            pltpu.VMEM((PER_B, FIELDS), jnp.int32),
            pltpu.VMEM((NBUF, CHUNK_B, FIELDS, EMBED_DIM), jnp.float32),
        ]
        + [pltpu.SemaphoreType.DMA] * (2 * NBUF),
        compiler_params=pltpu.CompilerParams(use_tc_tiling_on_sc=False),
    )
    def gather_kernel(table_hbm, idx_hbm, out_hbm, idx_v, rows_v, *sems):
        gsem = sems[:NBUF]
        osem = sems[NBUF:]
        wid = lax.axis_index("s") * NC + lax.axis_index("c")
        pltpu.sync_copy(idx_hbm.at[wid], idx_v)

        def start_gather(c, b):
            for j in range(CHUNK_B):
                pltpu.async_copy(
                    table_hbm.at[idx_v.at[c * CHUNK_B + j]],
                    rows_v.at[b].at[j],
                    gsem[b],
                )

        def wait_gather(b):
            for j in range(CHUNK_B):
                pltpu.make_async_copy(
                    table_hbm.at[idx_v.at[j]],
                    rows_v.at[b].at[j],
                    gsem[b],
                ).wait()

        def start_out(c, b):
            pltpu.async_copy(
                rows_v.at[b],
                out_hbm.at[pl.ds(wid * PER_B + c * CHUNK_B, CHUNK_B)],
                osem[b],
            )

        def wait_out(b):
            pltpu.make_async_copy(
                rows_v.at[b],
                out_hbm.at[pl.ds(wid * PER_B, CHUNK_B)],
                osem[b],
            ).wait()

        for b in range(NBUF):
            start_gather(jnp.int32(b), b)

        def body(g, _):
            for b in range(NBUF):
                c = g * NBUF + b
                wait_gather(b)
                start_out(c, b)
            for b in range(NBUF):
                c_next = (g + 1) * NBUF + b
                wait_out(b)
                start_gather(c_next, b)
            return 0

        lax.fori_loop(0, N_MAIN, body, 0)

        for b in range(NBUF):
            c = N_MAIN * NBUF + b
            wait_gather(b)
            start_out(jnp.int32(c), b)
        for b in range(NBUF):
            wait_out(b)

    return gather_kernel


_gather = _make_gather()


def kernel(severity_ids, weight):
    idx = severity_ids.reshape(NW, PER_B, FIELDS).astype(jnp.int32)
    return _gather(weight, idx)
